# baseline (speedup 1.0000x reference)
import functools
import math

import jax
import jax.numpy as jnp
from jax import lax
from jax.experimental import pallas as pl
from jax.experimental.pallas import tpu as pltpu
from jax.experimental.pallas import tpu_sc as plsc

D_MODEL = 64
SCALE = math.sqrt(D_MODEL)

_INFO = plsc.get_sparse_core_info()
_NC, _NS, _L = _INFO.num_cores, _INFO.num_subcores, _INFO.num_lanes
_NW = _NC * _NS

CHUNK = 512
SUBGATHER = 128


def _make_emb(B: int):
    b_per_w = B // _NW
    nchunks = b_per_w // CHUNK
    n_sub = CHUNK // SUBGATHER
    vregs_per_row = D_MODEL // _L

    mesh = plsc.VectorSubcoreMesh(core_axis_name="c", subcore_axis_name="s")

    @functools.partial(
        pl.kernel,
        mesh=mesh,
        out_type=jax.ShapeDtypeStruct((B, D_MODEL), jnp.float32),
        scratch_types=[
            pltpu.VMEM((CHUNK,), jnp.int32),
            pltpu.VMEM((CHUNK, D_MODEL), jnp.float32),
            pltpu.SemaphoreType.DMA,
        ],
        compiler_params=pltpu.CompilerParams(use_tc_tiling_on_sc=False),
    )
    def emb(x_hbm, lut_hbm, out_hbm, idx_v, rows_v, sem):
        wid = lax.axis_index("s") * _NC + lax.axis_index("c")
        base = wid * b_per_w

        def chunk_body(ci, carry):
            row0 = base + ci * CHUNK
            pltpu.sync_copy(x_hbm.at[pl.ds(row0, CHUNK)], idx_v)
            for k in range(n_sub):
                pltpu.async_copy(
                    lut_hbm.at[idx_v.at[pl.ds(k * SUBGATHER, SUBGATHER)]],
                    rows_v.at[pl.ds(k * SUBGATHER, SUBGATHER)],
                    sem,
                )
            for k in range(n_sub):
                pltpu.make_async_copy(
                    lut_hbm.at[idx_v.at[pl.ds(k * SUBGATHER, SUBGATHER)]],
                    rows_v.at[pl.ds(k * SUBGATHER, SUBGATHER)],
                    sem,
                ).wait()

            def scale_body(r, c):
                for j in range(vregs_per_row):
                    sl = pl.ds(j * _L, _L)
                    rows_v[r, sl] = rows_v[r, sl] * jnp.float32(SCALE)
                return c

            lax.fori_loop(0, CHUNK, scale_body, 0)
            pltpu.sync_copy(rows_v, out_hbm.at[pl.ds(row0, CHUNK)])
            return carry

        lax.fori_loop(0, nchunks, chunk_body, 0)

    return emb


def kernel(x, lut):
    B = x.shape[0] * x.shape[1]
    out = _make_emb(B)(x.reshape(B), lut)
    return out.reshape(x.shape + (D_MODEL,))


# speedup vs baseline: 1.1415x; 1.1415x over previous
import functools
import math

import jax
import jax.numpy as jnp
from jax import lax
from jax.experimental import pallas as pl
from jax.experimental.pallas import tpu as pltpu
from jax.experimental.pallas import tpu_sc as plsc

D_MODEL = 64
SCALE = math.sqrt(D_MODEL)

_INFO = plsc.get_sparse_core_info()
_NC, _NS, _L = _INFO.num_cores, _INFO.num_subcores, _INFO.num_lanes
_NW = _NC * _NS

CHUNK = 512
SUBGATHER = 128


def _make_emb(B: int):
    b_per_w = B // _NW
    nchunks = b_per_w // CHUNK
    n_sub = CHUNK // SUBGATHER
    vregs_per_row = D_MODEL // _L
    assert nchunks % 2 == 0 and nchunks >= 2

    mesh = plsc.VectorSubcoreMesh(core_axis_name="c", subcore_axis_name="s")

    @functools.partial(
        pl.kernel,
        mesh=mesh,
        out_type=jax.ShapeDtypeStruct((B, D_MODEL), jnp.float32),
        scratch_types=[
            pltpu.VMEM((b_per_w,), jnp.int32),
            pltpu.VMEM((2, CHUNK, D_MODEL), jnp.float32),
            pltpu.SemaphoreType.DMA,
            pltpu.SemaphoreType.DMA,
        ],
        compiler_params=pltpu.CompilerParams(use_tc_tiling_on_sc=False),
    )
    def emb(x_hbm, lut_hbm, out_hbm, idx_v, rows_v, sem_g, sem_s):
        wid = lax.axis_index("s") * _NC + lax.axis_index("c")
        base = wid * b_per_w
        pltpu.sync_copy(x_hbm.at[pl.ds(base, b_per_w)], idx_v)

        def fire_gather(ci, buf):
            for k in range(n_sub):
                pltpu.async_copy(
                    lut_hbm.at[idx_v.at[pl.ds(ci * CHUNK + k * SUBGATHER,
                                              SUBGATHER)]],
                    rows_v.at[buf].at[pl.ds(k * SUBGATHER, SUBGATHER)],
                    sem_g,
                )

        def wait_gather(ci, buf):
            for k in range(n_sub):
                pltpu.make_async_copy(
                    lut_hbm.at[idx_v.at[pl.ds(ci * CHUNK + k * SUBGATHER,
                                              SUBGATHER)]],
                    rows_v.at[buf].at[pl.ds(k * SUBGATHER, SUBGATHER)],
                    sem_g,
                ).wait()

        fire_gather(0, 0)

        @pl.loop(0, nchunks, step=2)
        def _outer(ci0):
            for b in range(2):
                ci = ci0 + b
                nb = 1 - b

                @pl.when(ci >= 1)
                def _wait_prev_store():
                    pltpu.make_async_copy(
                        rows_v.at[nb],
                        out_hbm.at[pl.ds(base + (ci - 1) * CHUNK, CHUNK)],
                        sem_s,
                    ).wait()

                @pl.when(ci + 1 < nchunks)
                def _fire_next_gather():
                    fire_gather(ci + 1, nb)

                wait_gather(ci, b)

                @pl.loop(0, CHUNK, unroll=8)
                def _scale(r):
                    for j in range(vregs_per_row):
                        sl = pl.ds(j * _L, _L)
                        rows_v[b, r, sl] = rows_v[b, r, sl] * jnp.float32(SCALE)

                pltpu.async_copy(
                    rows_v.at[b],
                    out_hbm.at[pl.ds(base + ci * CHUNK, CHUNK)],
                    sem_s,
                )

        pltpu.make_async_copy(
            rows_v.at[1],
            out_hbm.at[pl.ds(base + (nchunks - 1) * CHUNK, CHUNK)],
            sem_s,
        ).wait()

    return emb


def kernel(x, lut):
    B = x.shape[0] * x.shape[1]
    out = _make_emb(B)(x.reshape(B), lut)
    return out.reshape(x.shape + (D_MODEL,))
